# baseline (speedup 1.0000x reference)
import functools

import jax
import jax.numpy as jnp
from jax import lax
from jax.experimental import pallas as pl
from jax.experimental.pallas import tpu as pltpu
from jax.experimental.pallas import tpu_sc as plsc

N_ROWS = 100000
N_FEAT = 128
N_SEG = 512
NC = 2
NS = 16
COLS = N_FEAT // NC
NQ = COLS // 16
SEG_PER_TILE = N_SEG // NS
CHUNK = 128
GROUP = 16
NBUF = 3
ROWS_MAIN = 6256
ROWS_LAST = N_ROWS - (NS - 1) * ROWS_MAIN
NFULL = ROWS_LAST // CHUNK
REM_MAIN = ROWS_MAIN - NFULL * CHUNK
REM_LAST = ROWS_LAST - NFULL * CHUNK

_mesh = plsc.VectorSubcoreMesh(core_axis_name="c", subcore_axis_name="s")


def _tree_sum(vals):
    vals = list(vals)
    while len(vals) > 1:
        nxt = [vals[i] + vals[i + 1] for i in range(0, len(vals) - 1, 2)]
        if len(vals) % 2:
            nxt.append(vals[-1])
        vals = nxt
    return vals[0]


@functools.partial(
    pl.kernel,
    out_type=jax.ShapeDtypeStruct((N_SEG, N_FEAT), jnp.float32),
    mesh=_mesh,
    scratch_types=[
        pltpu.VMEM_SHARED((N_SEG, COLS), jnp.float32),
        pltpu.VMEM((N_SEG, COLS), jnp.float32),
        pltpu.VMEM((NBUF, CHUNK, COLS), jnp.float32),
        pltpu.VMEM((NBUF, CHUNK), jnp.int32),
        pltpu.VMEM((1, GROUP), jnp.int32),
        pltpu.VMEM((N_SEG // CHUNK, CHUNK), jnp.int32),
    ] + [pltpu.SemaphoreType.DMA] * (2 * NBUF),
    compiler_params=pltpu.CompilerParams(use_tc_tiling_on_sc=False,
                                         needs_layout_passes=False),
)
def _sc_segment_sum(x_hbm, b_hbm, out_hbm, acc_sp, acc_t, xbuf, idxbuf,
                    idxw, idbuf, *sems):
    semx = sems[0:NBUF]
    semi = sems[NBUF:2 * NBUF]
    c = lax.axis_index("c")
    s = lax.axis_index("s")
    col0 = c * COLS
    base = s * ROWS_MAIN

    zvec = jnp.zeros((16,), jnp.float32)
    def _zacc(i, carry):
        for q in range(NQ):
            acc_t[i, 16 * q:16 * q + 16] = zvec
        return carry
    lax.fori_loop(0, N_SEG, _zacc, 0)
    pltpu.sync_copy(acc_t.at[pl.ds(0, SEG_PER_TILE)],
                    acc_sp.at[pl.ds(s * SEG_PER_TILE, SEG_PER_TILE)])
    lane = lax.iota(jnp.int32, 16)
    for r in range(N_SEG // CHUNK):
        for q in range(CHUNK // 16):
            idbuf[r, 16 * q:16 * q + 16] = lane + (r * CHUNK + 16 * q)
    plsc.subcore_barrier()

    def load_descs(jj, b):
        start = base + jj * CHUNK
        return (
            pltpu.make_async_copy(b_hbm.at[pl.ds(start, CHUNK)],
                                  idxbuf.at[b], semi[b]),
            pltpu.make_async_copy(
                x_hbm.at[pl.ds(start, CHUNK), pl.ds(col0, COLS)],
                xbuf.at[b], semx[b]),
        )

    def window_op(b, g0, row0):
        idv = idxbuf[b, g0:g0 + GROUP]
        id_first = idv[0]
        id_last = idv[GROUP - 1]

        @pl.when(id_first == id_last)
        def _():
            for q in range(NQ):
                ssum = _tree_sum([xbuf[b, g0 + i, 16 * q:16 * q + 16]
                                  for i in range(GROUP)])
                plsc.addupdate(acc_t.at[id_first, pl.ds(16 * q, 16)], ssum)

        @pl.when(id_first != id_last)
        def _():
            pltpu.sync_copy(b_hbm.at[pl.ds(row0, GROUP)], idxw.at[0])
            pltpu.sync_copy(xbuf.at[b, pl.ds(g0, GROUP)],
                            acc_sp.at[idxw.at[0]], add=True)

    def compute_chunk(b, jj):
        start = base + jj * CHUNK
        for w in range(CHUNK // GROUP):
            window_op(b, GROUP * w, start + GROUP * w)

    for b in range(2):
        for d in load_descs(b, b):
            d.start()

    def pipe(j, carry):
        for t in range(NBUF):
            jj = NBUF * j + t
            for d in load_descs(jj, t):
                d.wait()

            @pl.when(jj + 2 < NFULL)
            def _():
                for d in load_descs(jj + 2, (t + 2) % NBUF):
                    d.start()

            compute_chunk(t, jj)
        return carry
    lax.fori_loop(0, NFULL // NBUF, pipe, 0)

    rem_start = base + NFULL * CHUNK

    def rem_chunk(nrows):
        pltpu.sync_copy(b_hbm.at[pl.ds(rem_start, nrows)],
                        idxbuf.at[0, pl.ds(0, nrows)])
        pltpu.sync_copy(x_hbm.at[pl.ds(rem_start, nrows), pl.ds(col0, COLS)],
                        xbuf.at[0, pl.ds(0, nrows)])
        for w in range(nrows // GROUP):
            window_op(0, GROUP * w, rem_start + GROUP * w)

    @pl.when(s < NS - 1)
    def _():
        rem_chunk(REM_MAIN)

    @pl.when(s == NS - 1)
    def _():
        rem_chunk(REM_LAST)

    for r in range(N_SEG // CHUNK):
        pltpu.sync_copy(acc_t.at[pl.ds(r * CHUNK, CHUNK)],
                        acc_sp.at[idbuf.at[r]], add=True)

    plsc.subcore_barrier()
    pltpu.sync_copy(acc_sp.at[pl.ds(s * SEG_PER_TILE, SEG_PER_TILE)],
                    out_hbm.at[pl.ds(s * SEG_PER_TILE, SEG_PER_TILE),
                               pl.ds(col0, COLS)])


def kernel(x, batch):
    return _sc_segment_sum(x, batch.astype(jnp.int32))


# speedup vs baseline: 1.8555x; 1.8555x over previous
import functools

import jax
import jax.numpy as jnp
from jax import lax
from jax.experimental import pallas as pl
from jax.experimental.pallas import tpu as pltpu
from jax.experimental.pallas import tpu_sc as plsc

N_ROWS = 100000
N_FEAT = 128
N_SEG = 512

R_SC = 35840
R_TC = N_ROWS - R_SC

NC = 2
NS = 16
COLS = N_FEAT // NC
SEG_PER_TILE = N_SEG // NS
CHUNK = 128
NBUF = 4
PER_TILE = R_SC // NS
NFULL = PER_TILE // CHUNK
REM = PER_TILE - NFULL * CHUNK

BLK = 1024
NBLK = -(-R_TC // BLK)
R_TC_PAD = NBLK * BLK

_mesh = plsc.VectorSubcoreMesh(core_axis_name="c", subcore_axis_name="s")


@functools.partial(
    pl.kernel,
    out_type=jax.ShapeDtypeStruct((N_SEG, N_FEAT), jnp.float32),
    mesh=_mesh,
    scratch_types=[
        pltpu.VMEM_SHARED((N_SEG, COLS), jnp.float32),
        pltpu.VMEM((NBUF, CHUNK, COLS), jnp.float32),
        pltpu.VMEM((NBUF, CHUNK), jnp.int32),
    ] + [pltpu.SemaphoreType.DMA] * (3 * NBUF),
    compiler_params=pltpu.CompilerParams(use_tc_tiling_on_sc=False),
)
def _sc_segment_sum(x_hbm, b_hbm, out_hbm, acc, xbuf, idxbuf, *sems):
    semx = sems[0:NBUF]
    semi = sems[NBUF:2 * NBUF]
    sems_ = sems[2 * NBUF:3 * NBUF]
    c = lax.axis_index("c")
    s = lax.axis_index("s")
    col0 = c * COLS
    base = R_TC + s * PER_TILE

    zvec = jnp.zeros((16,), jnp.float32)
    def _zrow(i, carry):
        for q in range(COLS // 16):
            xbuf[0, i, 16 * q:16 * q + 16] = zvec
        return carry
    lax.fori_loop(0, CHUNK, _zrow, 0)
    zidx = jnp.zeros((16,), jnp.int32)
    for q in range(CHUNK // 16):
        idxbuf[0, 16 * q:16 * q + 16] = zidx

    pltpu.sync_copy(xbuf.at[0, pl.ds(0, SEG_PER_TILE)],
                    acc.at[pl.ds(s * SEG_PER_TILE, SEG_PER_TILE)])
    plsc.subcore_barrier()

    def load_descs(jj, b):
        start = base + jj * CHUNK
        return (
            pltpu.make_async_copy(b_hbm.at[pl.ds(start, CHUNK)],
                                  idxbuf.at[b], semi[b]),
            pltpu.make_async_copy(
                x_hbm.at[pl.ds(start, CHUNK), pl.ds(col0, COLS)],
                xbuf.at[b], semx[b]),
        )

    def start_scatter(b):
        pltpu.async_copy(xbuf.at[b], acc.at[idxbuf.at[b]], sems_[b],
                         add=True)

    def wait_scatter(b):
        pltpu.make_async_copy(xbuf.at[b], acc.at[pl.ds(0, CHUNK)],
                              sems_[b]).wait()

    rem_start = base + NFULL * CHUNK
    pltpu.sync_copy(b_hbm.at[pl.ds(rem_start, REM)],
                    idxbuf.at[0, pl.ds(0, REM)])
    pltpu.sync_copy(x_hbm.at[pl.ds(rem_start, REM), pl.ds(col0, COLS)],
                    xbuf.at[0, pl.ds(0, REM)])
    pltpu.sync_copy(xbuf.at[0], acc.at[idxbuf.at[0]], add=True)

    for b in range(2):
        for d in load_descs(b, b):
            d.start()

    def step(jj, b):
        for d in load_descs(jj, b):
            d.wait()
        start_scatter(b)
        nxt = (b + 2) % NBUF

        @pl.when((jj >= 2) & (jj + 2 < NFULL))
        def _():
            wait_scatter(nxt)
            for d in load_descs(jj + 2, nxt):
                d.start()

        @pl.when(jj < 2)
        def _():
            for d in load_descs(jj + 2, nxt):
                d.start()

    def pipe(j, carry):
        for b in range(NBUF):
            step(NBUF * j + b, b)
        return carry
    lax.fori_loop(0, NFULL // NBUF, pipe, 0)
    for jj in range(NFULL - NFULL % NBUF, NFULL):
        step(jj, jj % NBUF)
    for b in range(NBUF):
        wait_scatter((NFULL - NBUF + b) % NBUF)

    plsc.subcore_barrier()
    pltpu.sync_copy(acc.at[pl.ds(s * SEG_PER_TILE, SEG_PER_TILE)],
                    out_hbm.at[pl.ds(s * SEG_PER_TILE, SEG_PER_TILE),
                               pl.ds(col0, COLS)])


def _tc_body(bid_ref, x_ref, out_ref):
    pid = pl.program_id(0)
    ids = bid_ref[0, 0, :]
    one_hot = (ids[:, None]
               == lax.broadcasted_iota(jnp.int32, (BLK, N_SEG), 1)
               ).astype(jnp.float32)
    part = lax.dot_general(one_hot, x_ref[...], (((0,), (0,)), ((), ())),
                           preferred_element_type=jnp.float32)

    @pl.when(pid == 0)
    def _():
        out_ref[...] = part

    @pl.when(pid != 0)
    def _():
        out_ref[...] = out_ref[...] + part


_tc_call = pl.pallas_call(
    _tc_body,
    grid=(NBLK,),
    in_specs=[pl.BlockSpec((1, 1, BLK), lambda j: (j, 0, 0)),
              pl.BlockSpec((BLK, N_FEAT), lambda j: (j, 0))],
    out_specs=pl.BlockSpec((N_SEG, N_FEAT), lambda j: (0, 0)),
    out_shape=jax.ShapeDtypeStruct((N_SEG, N_FEAT), jnp.float32),
)


def _add_body(a_ref, b_ref, o_ref):
    o_ref[...] = a_ref[...] + b_ref[...]


_add_call = pl.pallas_call(
    _add_body,
    out_shape=jax.ShapeDtypeStruct((N_SEG, N_FEAT), jnp.float32),
)


def kernel(x, batch):
    batch32 = batch.astype(jnp.int32)
    out_sc = _sc_segment_sum(x, batch32)
    ids_tc = jnp.concatenate(
        [batch32[:R_TC],
         jnp.full((R_TC_PAD - R_TC,), -1, jnp.int32)]).reshape(NBLK, 1, BLK)
    out_tc = _tc_call(ids_tc, x)
    return _add_call(out_sc, out_tc)
